# baseline (speedup 1.0000x reference)
import functools

import jax
import jax.numpy as jnp
from jax.experimental import pallas as pl
from jax.experimental.pallas import tpu as pltpu
from jax.experimental.pallas import tpu_sc as plsc

_B = 16384
_D = 64
_NW = 32
_RPW = _B // _NW
_CH = 128
_NCH = _RPW // _CH


def _sc_gather(uid, iid, user_table, item_table):
    mesh = plsc.VectorSubcoreMesh(core_axis_name="core",
                                  subcore_axis_name="subcore")

    @functools.partial(
        pl.kernel,
        out_type=[pltpu.HBM((_B, _D), jnp.float32),
                  pltpu.HBM((_B, _D), jnp.float32)],
        mesh=mesh,
        scratch_types=[
            pltpu.VMEM((_RPW,), jnp.int32),
            pltpu.VMEM((_RPW,), jnp.int32),
            pltpu.VMEM((_CH, _D), jnp.float32),
            pltpu.VMEM((_CH, _D), jnp.float32),
            pltpu.SemaphoreType.DMA,
            pltpu.SemaphoreType.DMA,
        ],
    )
    def k(uid_hbm, iid_hbm, ut_hbm, it_hbm, u_hbm, v_hbm,
          uidx_s, iidx_s, ubuf, vbuf, usem, vsem):
        wid = jax.lax.axis_index("subcore") * 2 + jax.lax.axis_index("core")
        base = wid * _RPW
        pltpu.sync_copy(uid_hbm.at[pl.ds(base, _RPW)], uidx_s)
        pltpu.sync_copy(iid_hbm.at[pl.ds(base, _RPW)], iidx_s)

        @pl.loop(0, _NCH)
        def _(g):
            c = g * _CH

            @pl.loop(0, _CH // 16)
            def _(t):
                uvec = uidx_s[pl.ds(c + t * 16, 16)]
                ivec = iidx_s[pl.ds(c + t * 16, 16)]
                for j in range(16):
                    pltpu.async_copy(ut_hbm.at[pl.ds(uvec[j], 1)],
                                     ubuf.at[pl.ds(t * 16 + j, 1)], usem)
                    pltpu.async_copy(it_hbm.at[pl.ds(ivec[j], 1)],
                                     vbuf.at[pl.ds(t * 16 + j, 1)], vsem)

            pltpu.make_async_copy(ut_hbm.at[pl.ds(0, _CH)], ubuf, usem).wait()
            pltpu.make_async_copy(it_hbm.at[pl.ds(0, _CH)], vbuf, vsem).wait()
            pltpu.sync_copy(ubuf, u_hbm.at[pl.ds(base + c, _CH)])
            pltpu.sync_copy(vbuf, v_hbm.at[pl.ds(base + c, _CH)])

    return k(uid, iid, user_table, item_table)


_TT = 2048


def _tr_body(src_ref, dst_ref):
    eye = jnp.eye(_D, dtype=jnp.float32)
    dst_ref[:] = jax.lax.dot_general(
        src_ref[:], eye, (((0,), (0,)), ((), ())),
        preferred_element_type=jnp.float32)


def _tc_transpose(t_tr, n):
    return pl.pallas_call(
        _tr_body,
        grid=(pl.cdiv(n, _TT),),
        in_specs=[pl.BlockSpec((_D, _TT), lambda i: (0, i))],
        out_specs=pl.BlockSpec((_TT, _D), lambda i: (i, 0)),
        out_shape=jax.ShapeDtypeStruct((n, _D), jnp.float32),
    )(t_tr)


def _bn_relu(h):
    one = jnp.ones((1, _B), jnp.float32)
    s = jnp.dot(one, h, preferred_element_type=jnp.float32)
    q = jnp.dot(one, h * h, preferred_element_type=jnp.float32)
    mean = s * (1.0 / _B)
    var = q * (1.0 / _B) - mean * mean
    a = jax.lax.rsqrt(var + 1e-5)
    return jnp.maximum(h * a - mean * a, 0.0)


def _mlp_body(u_ref, v_ref, w1a_ref, w1b_ref, b1_ref, w2_ref, b2_ref,
              w3_ref, b3_ref, w4_ref, b4_ref, out_ref):
    hp = jnp.float32
    h = (jnp.dot(u_ref[:], w1a_ref[:], preferred_element_type=hp)
         + jnp.dot(v_ref[:], w1b_ref[:], preferred_element_type=hp)
         + b1_ref[:])
    h = _bn_relu(h)
    h = jnp.dot(h, w2_ref[:], preferred_element_type=hp) + b2_ref[:]
    h = _bn_relu(h)
    h = jnp.dot(h, w3_ref[:], preferred_element_type=hp) + b3_ref[:]
    h = _bn_relu(h)
    z = jnp.dot(h, w4_ref[:], preferred_element_type=hp) + b4_ref[:]
    out_ref[:] = jax.nn.sigmoid(z) * 5.0 + 1.0


def _tc_mlp(u, v, W1a, W1b, b1, W2, b2, W3, b3, W4, b4):
    return pl.pallas_call(
        _mlp_body,
        out_shape=jax.ShapeDtypeStruct((_B, 1), jnp.float32),
        compiler_params=pltpu.CompilerParams(vmem_limit_bytes=67108864),
    )(u, v, W1a, W1b, b1, W2, b2, W3, b3, W4, b4)


def kernel(user_id, item_id, user_table, item_table,
           W1, b1, W2, b2, W3, b3, W4, b4):
    ut_rm = _tc_transpose(user_table.T, user_table.shape[0])
    it_rm = _tc_transpose(item_table.T, item_table.shape[0])
    u, v = _sc_gather(user_id, item_id, ut_rm, it_rm)
    return _tc_mlp(u, v, W1[:_D], W1[_D:], b1.reshape(1, -1),
                   W2, b2.reshape(1, -1), W3, b3.reshape(1, -1),
                   W4, b4.reshape(1, -1))


# speedup vs baseline: 1.3712x; 1.3712x over previous
import functools

import jax
import jax.numpy as jnp
from jax.experimental import pallas as pl
from jax.experimental.pallas import tpu as pltpu
from jax.experimental.pallas import tpu_sc as plsc

_B = 16384
_D = 64
_NW = 32
_RPW = _B // _NW
_CH = 128
_NCH = _RPW // _CH


def _sc_gather(uid, iid, user_table, item_table):
    mesh = plsc.VectorSubcoreMesh(core_axis_name="core",
                                  subcore_axis_name="subcore")

    @functools.partial(
        pl.kernel,
        out_type=[pltpu.HBM((_B, _D), jnp.float32),
                  pltpu.HBM((_B, _D), jnp.float32)],
        mesh=mesh,
        scratch_types=[
            pltpu.VMEM((_RPW,), jnp.int32),
            pltpu.VMEM((_RPW,), jnp.int32),
            pltpu.VMEM((_CH, _D), jnp.float32),
            pltpu.VMEM((_CH, _D), jnp.float32),
            pltpu.SemaphoreType.DMA,
            pltpu.SemaphoreType.DMA,
        ],
    )
    def k(uid_hbm, iid_hbm, ut_hbm, it_hbm, u_hbm, v_hbm,
          uidx_s, iidx_s, ubuf, vbuf, usem, vsem):
        wid = jax.lax.axis_index("subcore") * 2 + jax.lax.axis_index("core")
        base = wid * _RPW
        pltpu.sync_copy(uid_hbm.at[pl.ds(base, _RPW)], uidx_s)
        pltpu.sync_copy(iid_hbm.at[pl.ds(base, _RPW)], iidx_s)

        @pl.loop(0, _NCH)
        def _(g):
            c = g * _CH

            @pl.loop(0, _CH // 16)
            def _(t):
                uvec = uidx_s[pl.ds(c + t * 16, 16)]
                ivec = iidx_s[pl.ds(c + t * 16, 16)]
                for j in range(16):
                    pltpu.async_copy(ut_hbm.at[pl.ds(uvec[j], 1)],
                                     ubuf.at[pl.ds(t * 16 + j, 1)], usem)
                    pltpu.async_copy(it_hbm.at[pl.ds(ivec[j], 1)],
                                     vbuf.at[pl.ds(t * 16 + j, 1)], vsem)

            pltpu.make_async_copy(ut_hbm.at[pl.ds(0, _CH)], ubuf, usem).wait()
            pltpu.make_async_copy(it_hbm.at[pl.ds(0, _CH)], vbuf, vsem).wait()
            pltpu.sync_copy(ubuf, u_hbm.at[pl.ds(base + c, _CH)])
            pltpu.sync_copy(vbuf, v_hbm.at[pl.ds(base + c, _CH)])

    return k(uid, iid, user_table, item_table)


def _bn_relu(h):
    one = jnp.ones((1, _B), jnp.float32)
    s = jnp.dot(one, h, preferred_element_type=jnp.float32)
    q = jnp.dot(one, h * h, preferred_element_type=jnp.float32)
    mean = s * (1.0 / _B)
    var = q * (1.0 / _B) - mean * mean
    a = jax.lax.rsqrt(var + 1e-5)
    return jnp.maximum(h * a - mean * a, 0.0)


def _mlp_body(u_ref, v_ref, w1a_ref, w1b_ref, b1_ref, w2_ref, b2_ref,
              w3_ref, b3_ref, w4_ref, b4_ref, out_ref):
    hp = jnp.float32
    h = (jnp.dot(u_ref[:], w1a_ref[:], preferred_element_type=hp)
         + jnp.dot(v_ref[:], w1b_ref[:], preferred_element_type=hp)
         + b1_ref[:])
    h = _bn_relu(h)
    h = jnp.dot(h, w2_ref[:], preferred_element_type=hp) + b2_ref[:]
    h = _bn_relu(h)
    h = jnp.dot(h, w3_ref[:], preferred_element_type=hp) + b3_ref[:]
    h = _bn_relu(h)
    z = jnp.dot(h, w4_ref[:], preferred_element_type=hp) + b4_ref[:]
    out_ref[:] = jax.nn.sigmoid(z) * 5.0 + 1.0


def _tc_mlp(u, v, W1a, W1b, b1, W2, b2, W3, b3, W4, b4):
    return pl.pallas_call(
        _mlp_body,
        out_shape=jax.ShapeDtypeStruct((_B, 1), jnp.float32),
        compiler_params=pltpu.CompilerParams(vmem_limit_bytes=67108864),
    )(u, v, W1a, W1b, b1, W2, b2, W3, b3, W4, b4)


def kernel(user_id, item_id, user_table, item_table,
           W1, b1, W2, b2, W3, b3, W4, b4):
    u, v = _sc_gather(user_id, item_id, user_table, item_table)
    return _tc_mlp(u, v, W1[:_D], W1[_D:], b1.reshape(1, -1),
                   W2, b2.reshape(1, -1), W3, b3.reshape(1, -1),
                   W4, b4.reshape(1, -1))
